# baseline (speedup 1.0000x reference)
import functools

import jax
import jax.numpy as jnp
from jax import lax
from jax.experimental import pallas as pl
from jax.experimental.pallas import tpu as pltpu
from jax.experimental.pallas import tpu_sc as plsc

_BLK = 128
_NCLS = 3


def _sc_body(n_rows, rows_per_w, flat_pre, idx_hbm, labels, score_out,
             pred_out, laboh_out, idx_v, c0_v, c1_v, c2_v, lab_v, score_v,
             pred_v, laboh_v, sem):
    nc = 2
    wid = lax.axis_index("s") * nc + lax.axis_index("c")
    iota = lax.iota(jnp.int32, 16)
    n_blk = rows_per_w // _BLK
    for b in range(n_blk):
        base = wid * rows_per_w + b * _BLK
        for j, cv in enumerate((c0_v, c1_v, c2_v)):
            pltpu.sync_copy(idx_hbm.at[pl.ds(j * n_rows + base, _BLK)],
                            idx_v)
            pltpu.async_copy(flat_pre.at[idx_v], cv, sem).wait()
        pltpu.sync_copy(labels.at[pl.ds(base, _BLK)], lab_v)
        for c in range(_BLK // 16):
            sl = pl.ds(c * 16, 16)
            v0 = c0_v[sl]
            v1 = c1_v[sl]
            v2 = c2_v[sl]
            m = jnp.maximum(v0, jnp.maximum(v1, v2))
            e0 = jnp.exp(v0 - m)
            e1 = jnp.exp(v1 - m)
            e2 = jnp.exp(v2 - m)
            inv = 1.0 / (e0 + e1 + e2)
            one = jnp.full((16,), 1.0, jnp.float32)
            zero = jnp.full((16,), 0.0, jnp.float32)
            f0 = (jnp.where(v0 >= v1, one, zero)
                  * jnp.where(v0 >= v2, one, zero))
            f1 = (one - f0) * jnp.where(v1 >= v2, one, zero)
            f2 = one - f0 - f1
            lab = lab_v[sl]
            for j, (yj, pj) in enumerate(((e0, f0), (e1, f1), (e2, f2))):
                osl = pl.ds(j * _BLK + c * 16, 16)
                score_v[osl] = yj * inv
                pred_v[osl] = pj
                laboh_v[osl] = jnp.where(lab == j, one, zero)
        for j, (vbuf, obuf) in enumerate(((score_v, score_out),
                                          (pred_v, pred_out),
                                          (laboh_v, laboh_out))):
            for jj in range(_NCLS):
                pltpu.sync_copy(vbuf.at[pl.ds(jj * _BLK, _BLK)],
                                obuf.at[pl.ds(jj * n_rows + base, _BLK)])


def kernel(pre, y_label, stage_name):
    n, d = pre.shape
    flat_pre = pre.reshape(n * d)
    col = jnp.arange(n, dtype=jnp.int32) * d
    idx = (col[None, :] + jnp.arange(_NCLS, dtype=jnp.int32)[:, None])
    idx = idx.reshape(_NCLS * n)
    labels = y_label.astype(jnp.int32)

    info = plsc.get_sparse_core_info()
    n_workers = info.num_cores * info.num_subcores
    rows_per_w = n // n_workers
    mesh = plsc.VectorSubcoreMesh(core_axis_name="c", subcore_axis_name="s")

    flat = jax.ShapeDtypeStruct((n * _NCLS,), jnp.float32)
    k = functools.partial(
        pl.kernel,
        out_type=(flat, flat, flat),
        mesh=mesh,
        scratch_types=[
            pltpu.VMEM((_BLK,), jnp.int32),
            pltpu.VMEM((_BLK,), jnp.float32),
            pltpu.VMEM((_BLK,), jnp.float32),
            pltpu.VMEM((_BLK,), jnp.float32),
            pltpu.VMEM((_BLK,), jnp.int32),
            pltpu.VMEM((_BLK * _NCLS,), jnp.float32),
            pltpu.VMEM((_BLK * _NCLS,), jnp.float32),
            pltpu.VMEM((_BLK * _NCLS,), jnp.float32),
            pltpu.SemaphoreType.DMA,
        ],
    )(functools.partial(_sc_body, n, rows_per_w))
    score, pred_oh, lab_oh = k(flat_pre, idx, labels)
    def _t(a):
        return a.reshape(_NCLS, n).T
    return (_t(score), _t(pred_oh), _t(lab_oh))
